# baseline (speedup 1.0000x reference)
import jax
import jax.numpy as jnp
from jax import lax
from jax.experimental import pallas as pl
from jax.experimental.pallas import tpu as pltpu
from jax.experimental.pallas import tpu_sc as plsc

_NC = 2
_NS = 16
_NW = _NC * _NS
_LANES = 16
_CROWS = 8
_UNROLL = 8


def _tc_body(inp_ref, targ_ref, sum_ref, cnt_ref):
    i = pl.program_id(0)

    @pl.when(i == 0)
    def _init():
        sum_ref[0] = 0.0
        cnt_ref[0] = 0.0

    t = targ_ref[...]
    x = inp_ref[...]
    mask = jnp.isnan(t)
    d = jnp.where(mask, 0.0, x - t)
    sum_ref[0] += jnp.sum(d * d)
    cnt_ref[0] += jnp.sum(jnp.where(mask, 0.0, 1.0))


def _tc_masked_mse(x, t, skip_rows=0):
    rows, cols = x.shape
    block_rows = 512
    grid = (rows - skip_rows) // block_rows
    blk_off = skip_rows // block_rows
    index_map = lambda i: (i + blk_off, 0)
    s, c = pl.pallas_call(
        _tc_body,
        grid=(grid,),
        in_specs=[
            pl.BlockSpec((block_rows, cols), index_map),
            pl.BlockSpec((block_rows, cols), index_map),
        ],
        out_specs=[
            pl.BlockSpec(memory_space=pltpu.SMEM),
            pl.BlockSpec(memory_space=pltpu.SMEM),
        ],
        out_shape=[
            jax.ShapeDtypeStruct((1,), jnp.float32),
            jax.ShapeDtypeStruct((1,), jnp.float32),
        ],
    )(x, t)
    return s[0], c[0]


def _sc_masked_mse(x, t, sc_rows=None):
    rows, cols = x.shape
    if sc_rows is None:
        sc_rows = rows
    rows_per_w = sc_rows // _NW
    nchunks = rows_per_w // _CROWS
    npairs = nchunks // 2

    def body(x_hbm, t_hbm, sum_hbm, cnt_hbm,
             xb0, xb1, tb0, tb1, ob_s, ob_c,
             sx0, sx1, st0, st1):
        cid = lax.axis_index("c")
        sid = lax.axis_index("s")
        wid = sid * _NC + cid
        base = wid * rows_per_w

        xbufs = (xb0, xb1)
        tbufs = (tb0, tb1)
        sxs = (sx0, sx1)
        sts = (st0, st1)

        def start(k, slot):
            off = base + k * _CROWS
            pltpu.make_async_copy(x_hbm.at[pl.ds(off, _CROWS)], xbufs[slot],
                                  sxs[slot]).start()
            pltpu.make_async_copy(t_hbm.at[pl.ds(off, _CROWS)], tbufs[slot],
                                  sts[slot]).start()

        def wait(slot):
            pltpu.make_async_copy(x_hbm.at[pl.ds(0, _CROWS)], xbufs[slot],
                                  sxs[slot]).wait()
            pltpu.make_async_copy(t_hbm.at[pl.ds(0, _CROWS)], tbufs[slot],
                                  sts[slot]).wait()

        def compute(slot, accs):
            xb = xbufs[slot]
            tb = tbufs[slot]

            def row(r, accs):
                xr = xb.at[r]
                tr = tb.at[r]

                def inner(j, carry):
                    a0, a1, c0, c1 = carry
                    jbase = j * (_LANES * _UNROLL)
                    for u in range(_UNROLL):
                        off = jbase + u * _LANES
                        xv = xr[pl.ds(off, _LANES)]
                        tv = tr[pl.ds(off, _LANES)]
                        d = xv - tv
                        d2 = d * d
                        m = d2 == d2
                        if u % 2 == 0:
                            a0 = a0 + jnp.where(m, d2, 0.0)
                            c0 = c0 + jnp.where(m, 1.0, 0.0)
                        else:
                            a1 = a1 + jnp.where(m, d2, 0.0)
                            c1 = c1 + jnp.where(m, 1.0, 0.0)
                    return a0, a1, c0, c1

                return lax.fori_loop(0, cols // (_LANES * _UNROLL), inner,
                                     accs)

            for r in range(_CROWS):
                accs = row(r, accs)
            return accs

        start(0, 0)
        zero = jnp.zeros((_LANES,), jnp.float32)

        def pair(p, carry):
            accs = carry
            start(2 * p + 1, 1)
            wait(0)
            accs = compute(0, accs)

            @pl.when(p + 1 < npairs)
            def _():
                start(2 * p + 2, 0)

            wait(1)
            accs = compute(1, accs)
            return accs

        a0, a1, c0, c1 = lax.fori_loop(0, npairs, pair,
                                       (zero, zero, zero, zero))
        ob_s[...] = a0 + a1
        ob_c[...] = c0 + c1
        pltpu.sync_copy(ob_s, sum_hbm.at[pl.ds(wid * _LANES, _LANES)])
        pltpu.sync_copy(ob_c, cnt_hbm.at[pl.ds(wid * _LANES, _LANES)])

    mesh = plsc.VectorSubcoreMesh(core_axis_name="c", subcore_axis_name="s",
                                  num_cores=_NC, num_subcores=_NS)
    s, c = pl.kernel(
        body,
        out_type=[
            jax.ShapeDtypeStruct((_NW * _LANES,), jnp.float32),
            jax.ShapeDtypeStruct((_NW * _LANES,), jnp.float32),
        ],
        mesh=mesh,
        scratch_types=[
            pltpu.VMEM((_CROWS, 2048), jnp.float32),
            pltpu.VMEM((_CROWS, 2048), jnp.float32),
            pltpu.VMEM((_CROWS, 2048), jnp.float32),
            pltpu.VMEM((_CROWS, 2048), jnp.float32),
            pltpu.VMEM((_LANES,), jnp.float32),
            pltpu.VMEM((_LANES,), jnp.float32),
            pltpu.SemaphoreType.DMA,
            pltpu.SemaphoreType.DMA,
            pltpu.SemaphoreType.DMA,
            pltpu.SemaphoreType.DMA,
        ],
    )(x, t)
    return jnp.sum(s), jnp.sum(c)


_SC_ROWS = 4096


def kernel(inp, targ):
    cols = inp.shape[-1]
    x = inp.reshape(-1, cols)
    t = targ.reshape(-1, cols)
    s2, c2 = _tc_masked_mse(x, t, _SC_ROWS)
    s1, c1 = _sc_masked_mse(x, t, _SC_ROWS)
    return (s1 + s2) / (c1 + c2)


# speedup vs baseline: 1.0079x; 1.0079x over previous
import jax
import jax.numpy as jnp
from jax import lax
from jax.experimental import pallas as pl
from jax.experimental.pallas import tpu as pltpu
from jax.experimental.pallas import tpu_sc as plsc

_NC = 2
_NS = 16
_NW = _NC * _NS
_LANES = 16
_CROWS = 8
_UNROLL = 8


def _tc_body(inp_ref, targ_ref, sum_ref, cnt_ref):
    i = pl.program_id(0)

    @pl.when(i == 0)
    def _init():
        sum_ref[0] = 0.0
        cnt_ref[0] = 0.0

    t = targ_ref[...]
    x = inp_ref[...]
    mask = jnp.isnan(t)
    d = jnp.where(mask, 0.0, x - t)
    sum_ref[0] += jnp.sum(d * d)
    cnt_ref[0] += jnp.sum(jnp.where(mask, 0.0, 1.0))


def _tc_masked_mse(x, t, skip_rows=0):
    rows, cols = x.shape
    block_rows = 512
    grid = (rows - skip_rows) // block_rows
    blk_off = skip_rows // block_rows
    index_map = lambda i: (i + blk_off, 0)
    s, c = pl.pallas_call(
        _tc_body,
        grid=(grid,),
        in_specs=[
            pl.BlockSpec((block_rows, cols), index_map),
            pl.BlockSpec((block_rows, cols), index_map),
        ],
        out_specs=[
            pl.BlockSpec(memory_space=pltpu.SMEM),
            pl.BlockSpec(memory_space=pltpu.SMEM),
        ],
        out_shape=[
            jax.ShapeDtypeStruct((1,), jnp.float32),
            jax.ShapeDtypeStruct((1,), jnp.float32),
        ],
    )(x, t)
    return s[0], c[0]


def _sc_masked_mse(x, t, sc_rows=None):
    rows, cols = x.shape
    if sc_rows is None:
        sc_rows = rows
    rows_per_w = sc_rows // _NW
    nchunks = rows_per_w // _CROWS
    npairs = nchunks // 2

    def body(x_hbm, t_hbm, sum_hbm, cnt_hbm,
             xb0, xb1, tb0, tb1, ob_s, ob_c,
             sx0, sx1, st0, st1):
        cid = lax.axis_index("c")
        sid = lax.axis_index("s")
        wid = sid * _NC + cid
        base = wid * rows_per_w

        xbufs = (xb0, xb1)
        tbufs = (tb0, tb1)
        sxs = (sx0, sx1)
        sts = (st0, st1)

        def start(k, slot):
            off = base + k * _CROWS
            pltpu.make_async_copy(x_hbm.at[pl.ds(off, _CROWS)], xbufs[slot],
                                  sxs[slot]).start()
            pltpu.make_async_copy(t_hbm.at[pl.ds(off, _CROWS)], tbufs[slot],
                                  sts[slot]).start()

        def wait(slot):
            pltpu.make_async_copy(x_hbm.at[pl.ds(0, _CROWS)], xbufs[slot],
                                  sxs[slot]).wait()
            pltpu.make_async_copy(t_hbm.at[pl.ds(0, _CROWS)], tbufs[slot],
                                  sts[slot]).wait()

        def compute(slot, accs):
            xb = xbufs[slot]
            tb = tbufs[slot]

            def row(r, accs):
                xr = xb.at[r]
                tr = tb.at[r]

                def inner(j, carry):
                    a0, a1, c0, c1 = carry
                    jbase = j * (_LANES * _UNROLL)
                    for u in range(_UNROLL):
                        off = jbase + u * _LANES
                        xv = xr[pl.ds(off, _LANES)]
                        tv = tr[pl.ds(off, _LANES)]
                        d = xv - tv
                        d2 = d * d
                        m = d2 == d2
                        if u % 2 == 0:
                            a0 = a0 + jnp.where(m, d2, 0.0)
                            c0 = c0 + jnp.where(m, 1.0, 0.0)
                        else:
                            a1 = a1 + jnp.where(m, d2, 0.0)
                            c1 = c1 + jnp.where(m, 1.0, 0.0)
                    return a0, a1, c0, c1

                return lax.fori_loop(0, cols // (_LANES * _UNROLL), inner,
                                     accs)

            for r in range(_CROWS):
                accs = row(r, accs)
            return accs

        start(0, 0)
        zero = jnp.zeros((_LANES,), jnp.float32)

        def pair(p, carry):
            accs = carry
            start(2 * p + 1, 1)
            wait(0)
            accs = compute(0, accs)

            @pl.when(p + 1 < npairs)
            def _():
                start(2 * p + 2, 0)

            wait(1)
            accs = compute(1, accs)
            return accs

        a0, a1, c0, c1 = lax.fori_loop(0, npairs, pair,
                                       (zero, zero, zero, zero))
        ob_s[...] = a0 + a1
        ob_c[...] = c0 + c1
        pltpu.sync_copy(ob_s, sum_hbm.at[pl.ds(wid * _LANES, _LANES)])
        pltpu.sync_copy(ob_c, cnt_hbm.at[pl.ds(wid * _LANES, _LANES)])

    mesh = plsc.VectorSubcoreMesh(core_axis_name="c", subcore_axis_name="s",
                                  num_cores=_NC, num_subcores=_NS)
    s, c = pl.kernel(
        body,
        out_type=[
            jax.ShapeDtypeStruct((_NW * _LANES,), jnp.float32),
            jax.ShapeDtypeStruct((_NW * _LANES,), jnp.float32),
        ],
        mesh=mesh,
        scratch_types=[
            pltpu.VMEM((_CROWS, 2048), jnp.float32),
            pltpu.VMEM((_CROWS, 2048), jnp.float32),
            pltpu.VMEM((_CROWS, 2048), jnp.float32),
            pltpu.VMEM((_CROWS, 2048), jnp.float32),
            pltpu.VMEM((_LANES,), jnp.float32),
            pltpu.VMEM((_LANES,), jnp.float32),
            pltpu.SemaphoreType.DMA,
            pltpu.SemaphoreType.DMA,
            pltpu.SemaphoreType.DMA,
            pltpu.SemaphoreType.DMA,
        ],
    )(x, t)
    return jnp.sum(s), jnp.sum(c)


_SC_ROWS = 4096


def kernel(inp, targ):
    cols = inp.shape[-1]
    x = inp.reshape(-1, cols)
    t = targ.reshape(-1, cols)
    s1, c1 = _sc_masked_mse(x, t, _SC_ROWS)
    s2, c2 = _tc_masked_mse(x, t, _SC_ROWS)
    return (s1 + s2) / (c1 + c2)


# speedup vs baseline: 1.0248x; 1.0168x over previous
import jax
import jax.numpy as jnp
from jax import lax
from jax.experimental import pallas as pl
from jax.experimental.pallas import tpu as pltpu
from jax.experimental.pallas import tpu_sc as plsc

_NC = 1
_NS = 16
_NW = _NC * _NS
_LANES = 16
_CROWS = 8
_UNROLL = 8


def _tc_body(inp_ref, targ_ref, sum_ref, cnt_ref):
    i = pl.program_id(0)

    @pl.when(i == 0)
    def _init():
        sum_ref[0] = 0.0
        cnt_ref[0] = 0.0

    t = targ_ref[...]
    x = inp_ref[...]
    mask = jnp.isnan(t)
    d = jnp.where(mask, 0.0, x - t)
    sum_ref[0] += jnp.sum(d * d)
    cnt_ref[0] += jnp.sum(jnp.where(mask, 0.0, 1.0))


def _tc_masked_mse(x, t, skip_rows=0):
    rows, cols = x.shape
    block_rows = 512
    grid = (rows - skip_rows) // block_rows
    blk_off = skip_rows // block_rows
    index_map = lambda i: (i + blk_off, 0)
    s, c = pl.pallas_call(
        _tc_body,
        grid=(grid,),
        in_specs=[
            pl.BlockSpec((block_rows, cols), index_map),
            pl.BlockSpec((block_rows, cols), index_map),
        ],
        out_specs=[
            pl.BlockSpec(memory_space=pltpu.SMEM),
            pl.BlockSpec(memory_space=pltpu.SMEM),
        ],
        out_shape=[
            jax.ShapeDtypeStruct((1,), jnp.float32),
            jax.ShapeDtypeStruct((1,), jnp.float32),
        ],
    )(x, t)
    return s[0], c[0]


def _sc_masked_mse(x, t, sc_rows=None):
    rows, cols = x.shape
    if sc_rows is None:
        sc_rows = rows
    rows_per_w = sc_rows // _NW
    nchunks = rows_per_w // _CROWS
    npairs = nchunks // 2

    def body(x_hbm, t_hbm, sum_hbm, cnt_hbm,
             xb0, xb1, tb0, tb1, ob_s, ob_c,
             sx0, sx1, st0, st1):
        cid = lax.axis_index("c")
        sid = lax.axis_index("s")
        wid = sid * _NC + cid
        base = wid * rows_per_w

        xbufs = (xb0, xb1)
        tbufs = (tb0, tb1)
        sxs = (sx0, sx1)
        sts = (st0, st1)

        def start(k, slot):
            off = base + k * _CROWS
            pltpu.make_async_copy(x_hbm.at[pl.ds(off, _CROWS)], xbufs[slot],
                                  sxs[slot]).start()
            pltpu.make_async_copy(t_hbm.at[pl.ds(off, _CROWS)], tbufs[slot],
                                  sts[slot]).start()

        def wait(slot):
            pltpu.make_async_copy(x_hbm.at[pl.ds(0, _CROWS)], xbufs[slot],
                                  sxs[slot]).wait()
            pltpu.make_async_copy(t_hbm.at[pl.ds(0, _CROWS)], tbufs[slot],
                                  sts[slot]).wait()

        def compute(slot, accs):
            xb = xbufs[slot]
            tb = tbufs[slot]

            def row(r, accs):
                xr = xb.at[r]
                tr = tb.at[r]

                def inner(j, carry):
                    a0, a1, c0, c1 = carry
                    jbase = j * (_LANES * _UNROLL)
                    for u in range(_UNROLL):
                        off = jbase + u * _LANES
                        xv = xr[pl.ds(off, _LANES)]
                        tv = tr[pl.ds(off, _LANES)]
                        d = xv - tv
                        d2 = d * d
                        m = d2 == d2
                        if u % 2 == 0:
                            a0 = a0 + jnp.where(m, d2, 0.0)
                            c0 = c0 + jnp.where(m, 1.0, 0.0)
                        else:
                            a1 = a1 + jnp.where(m, d2, 0.0)
                            c1 = c1 + jnp.where(m, 1.0, 0.0)
                    return a0, a1, c0, c1

                return lax.fori_loop(0, cols // (_LANES * _UNROLL), inner,
                                     accs)

            for r in range(_CROWS):
                accs = row(r, accs)
            return accs

        start(0, 0)
        zero = jnp.zeros((_LANES,), jnp.float32)

        def pair(p, carry):
            accs = carry
            start(2 * p + 1, 1)
            wait(0)
            accs = compute(0, accs)

            @pl.when(p + 1 < npairs)
            def _():
                start(2 * p + 2, 0)

            wait(1)
            accs = compute(1, accs)
            return accs

        a0, a1, c0, c1 = lax.fori_loop(0, npairs, pair,
                                       (zero, zero, zero, zero))
        ob_s[...] = a0 + a1
        ob_c[...] = c0 + c1
        pltpu.sync_copy(ob_s, sum_hbm.at[pl.ds(wid * _LANES, _LANES)])
        pltpu.sync_copy(ob_c, cnt_hbm.at[pl.ds(wid * _LANES, _LANES)])

    mesh = plsc.VectorSubcoreMesh(core_axis_name="c", subcore_axis_name="s",
                                  num_cores=_NC, num_subcores=_NS)
    s, c = pl.kernel(
        body,
        out_type=[
            jax.ShapeDtypeStruct((_NW * _LANES,), jnp.float32),
            jax.ShapeDtypeStruct((_NW * _LANES,), jnp.float32),
        ],
        mesh=mesh,
        scratch_types=[
            pltpu.VMEM((_CROWS, 2048), jnp.float32),
            pltpu.VMEM((_CROWS, 2048), jnp.float32),
            pltpu.VMEM((_CROWS, 2048), jnp.float32),
            pltpu.VMEM((_CROWS, 2048), jnp.float32),
            pltpu.VMEM((_LANES,), jnp.float32),
            pltpu.VMEM((_LANES,), jnp.float32),
            pltpu.SemaphoreType.DMA,
            pltpu.SemaphoreType.DMA,
            pltpu.SemaphoreType.DMA,
            pltpu.SemaphoreType.DMA,
        ],
    )(x, t)
    return jnp.sum(s), jnp.sum(c)


_SC_ROWS = 2048


def kernel(inp, targ):
    cols = inp.shape[-1]
    x = inp.reshape(-1, cols)
    t = targ.reshape(-1, cols)
    s1, c1 = _sc_masked_mse(x, t, _SC_ROWS)
    s2, c2 = _tc_masked_mse(x, t, _SC_ROWS)
    return (s1 + s2) / (c1 + c2)
